# Initial kernel scaffold; baseline (speedup 1.0000x reference)
import jax
import jax.numpy as jnp
from jax.experimental import pallas as pl


def kernel(x, minute_W, hour_W, weekday_W, day_W, month_W, year_W):
    raise NotImplementedError("write your pallas kernel here")


# speedup vs baseline: 10.0965x; 10.0965x over previous
import jax
import jax.numpy as jnp
from jax import lax
from jax.experimental import pallas as pl

B, S, D = 4096, 50, 128
MINUTE, HOUR, WEEKDAY, DAY, MONTH, YEAR = 60, 24, 7, 32, 12, 20
NTOK = B * S
VCAT = 160

_OFF_MINUTE = 0
_OFF_HOUR = 60
_OFF_WEEKDAY = 84
_OFF_DAY = 91
_OFF_MONTH = 123
_OFF_YEAR = 135

_TBLOCK = 2048


def _body(x_ref, w_ref, o_ref):
    xb = x_ref[...]
    iota = lax.broadcasted_iota(jnp.int32, (_TBLOCK, VCAT), 1)
    m = jnp.zeros((_TBLOCK, VCAT), jnp.float32)
    cols = [
        xb[:, 5] + _OFF_MINUTE,
        xb[:, 4] + _OFF_HOUR,
        xb[:, 2] + _OFF_WEEKDAY,
        xb[:, 3] + _OFF_DAY,
        xb[:, 1] + _OFF_MONTH,
        xb[:, 0] - 2024 + _OFF_YEAR,
    ]
    for c in cols:
        m = m + (iota == c[:, None]).astype(jnp.float32)
    o_ref[...] = jnp.dot(m, w_ref[...], preferred_element_type=jnp.float32)


def kernel(x, minute_W, hour_W, weekday_W, day_W, month_W, year_W):
    x2 = x.reshape(NTOK, 6).astype(jnp.int32)
    wcat = jnp.concatenate(
        [minute_W, hour_W, weekday_W, day_W, month_W, year_W,
         jnp.zeros((VCAT - 155, D), jnp.float32)], axis=0)
    grid = NTOK // _TBLOCK
    out = pl.pallas_call(
        _body,
        grid=(grid,),
        in_specs=[
            pl.BlockSpec((_TBLOCK, 6), lambda i: (i, 0)),
            pl.BlockSpec((VCAT, D), lambda i: (0, 0)),
        ],
        out_specs=pl.BlockSpec((_TBLOCK, D), lambda i: (i, 0)),
        out_shape=jax.ShapeDtypeStruct((NTOK, D), jnp.float32),
    )(x2, wcat)
    return out.reshape(B, S, D)


# speedup vs baseline: 10.1338x; 1.0037x over previous
import functools

import jax
import jax.numpy as jnp
from jax import lax
from jax.experimental import pallas as pl
from jax.experimental.pallas import tpu as pltpu
from jax.experimental.pallas import tpu_sc as plsc

B, S, D = 4096, 50, 128
MINUTE, HOUR, WEEKDAY, DAY, MONTH, YEAR = 60, 24, 7, 32, 12, 20
NTOK = B * S

NC, NS, L = 2, 16, 16
NW = NC * NS
TOK_PER_W = NTOK // NW
CH = 128
NCHUNK = TOK_PER_W // CH

V1 = MINUTE * WEEKDAY * YEAR
V2 = HOUR * DAY * MONTH
VCAT = V1 + V2

_mesh = plsc.VectorSubcoreMesh(core_axis_name="c", subcore_axis_name="s")


@functools.partial(
    pl.kernel,
    out_type=jax.ShapeDtypeStruct((NTOK, D), jnp.float32),
    mesh=_mesh,
    scratch_types=[
        pltpu.VMEM((6, CH), jnp.int32),
        pltpu.VMEM((CH,), jnp.int32),
        pltpu.VMEM((CH,), jnp.int32),
        pltpu.VMEM((CH, D), jnp.float32),
        pltpu.VMEM((CH, D), jnp.float32),
        pltpu.SemaphoreType.DMA,
        pltpu.SemaphoreType.DMA,
    ],
)
def _sc_embed(wc_hbm, xt_hbm, out_hbm, fld_v, idx1_v, idx2_v, bufA, bufB,
              semA, semB):
    wid = lax.axis_index("s") * NC + lax.axis_index("c")
    wbase = wid * TOK_PER_W

    def chunk_body(c, _):
        base = wbase + c * CH
        pltpu.sync_copy(xt_hbm.at[:, pl.ds(base, CH)], fld_v)
        for j in range(CH // L):
            sl = pl.ds(j * L, L)
            yr = fld_v[0, sl]
            mo = fld_v[1, sl]
            wd = fld_v[2, sl]
            dy = fld_v[3, sl]
            hr = fld_v[4, sl]
            mi = fld_v[5, sl]
            idx1_v[sl] = mi * (WEEKDAY * YEAR) + wd * YEAR + (yr - 2024)
            idx2_v[sl] = hr * (DAY * MONTH) + dy * MONTH + mo + V1
        cpA = pltpu.async_copy(wc_hbm.at[idx1_v], bufA, semA)
        cpB = pltpu.async_copy(wc_hbm.at[idx2_v], bufB, semB)
        cpA.wait()
        cpB.wait()

        def add_body(t, _):
            for cc in range(D // L):
                sl2 = pl.ds(cc * L, L)
                plsc.addupdate(bufA.at[t, sl2], bufB[t, sl2])
            return ()

        lax.fori_loop(0, CH, add_body, (), unroll=2)
        pltpu.sync_copy(bufA, out_hbm.at[pl.ds(base, CH), :])
        return ()

    lax.fori_loop(0, NCHUNK, chunk_body, ())


def kernel(x, minute_W, hour_W, weekday_W, day_W, month_W, year_W):
    w1 = (minute_W[:, None, None, :] + weekday_W[None, :, None, :]
          + year_W[None, None, :, :]).reshape(V1, D)
    w2 = (hour_W[:, None, None, :] + day_W[None, :, None, :]
          + month_W[None, None, :, :]).reshape(V2, D)
    wc = jnp.concatenate([w1, w2], axis=0)
    xt = x.reshape(NTOK, 6).astype(jnp.int32).T
    out = _sc_embed(wc, xt)
    return out.reshape(B, S, D)


# speedup vs baseline: 13.0411x; 1.2869x over previous
import functools

import jax
import jax.numpy as jnp
from jax import lax
from jax.experimental import pallas as pl
from jax.experimental.pallas import tpu as pltpu
from jax.experimental.pallas import tpu_sc as plsc

B, S, D = 4096, 50, 128
MINUTE, HOUR, WEEKDAY, DAY, MONTH, YEAR = 60, 24, 7, 32, 12, 20
NTOK = B * S
NF = 6

NC, NS, L = 2, 16, 16
NW = NC * NS
TOK_PER_W = NTOK // NW
CH = 128
NCHUNK = TOK_PER_W // CH

V1 = MINUTE * WEEKDAY * YEAR
V2 = HOUR * DAY * MONTH

_mesh = plsc.VectorSubcoreMesh(core_axis_name="c", subcore_axis_name="s")


@functools.partial(
    pl.kernel,
    out_type=jax.ShapeDtypeStruct((NTOK, D), jnp.float32),
    mesh=_mesh,
    scratch_types=[
        [pltpu.VMEM((NF, CH), jnp.int32)] * 2,
        [pltpu.VMEM((CH,), jnp.int32)] * 2,
        [pltpu.VMEM((CH,), jnp.int32)] * 2,
        [pltpu.VMEM((CH, D), jnp.float32)] * 2,
        [pltpu.VMEM((CH, D), jnp.float32)] * 2,
        [pltpu.SemaphoreType.DMA] * 2,
        [pltpu.SemaphoreType.DMA] * 2,
        [pltpu.SemaphoreType.DMA] * 2,
        [pltpu.SemaphoreType.DMA] * 2,
    ],
)
def _sc_embed(wc_hbm, xf_hbm, out_hbm, fld, idx1, idx2, bufA, bufB,
              semF, semGA, semGB, semO):
    wid = lax.axis_index("s") * NC + lax.axis_index("c")
    wbase = wid * TOK_PER_W
    iota = lax.iota(jnp.int32, L)

    def tok_base(k):
        return wbase + k * CH

    def fields_start(k, b):
        pltpu.async_copy(xf_hbm.at[:, pl.ds(tok_base(k), CH)],
                         fld[b], semF[b])

    def fields_wait(b):
        pltpu.make_async_copy(xf_hbm.at[:, pl.ds(0, CH)], fld[b],
                              semF[b]).wait()

    def idx_compute(b):
        for j in range(CH // L):
            sl = pl.ds(j * L, L)
            yr = fld[b][0, sl]
            mo = fld[b][1, sl]
            wd = fld[b][2, sl]
            dy = fld[b][3, sl]
            hr = fld[b][4, sl]
            mi = fld[b][5, sl]
            idx1[b][sl] = mi * (WEEKDAY * YEAR) + wd * YEAR + (yr - 2024)
            idx2[b][sl] = hr * (DAY * MONTH) + dy * MONTH + mo + V1

    def gathers_start(b):
        pltpu.async_copy(wc_hbm.at[idx1[b]], bufA[b], semGA[b])
        pltpu.async_copy(wc_hbm.at[idx2[b]], bufB[b], semGB[b])

    def gathers_wait(b):
        pltpu.make_async_copy(wc_hbm.at[idx1[b]], bufA[b], semGA[b]).wait()
        pltpu.make_async_copy(wc_hbm.at[idx2[b]], bufB[b], semGB[b]).wait()

    def accumulate(b):
        def add_body(t, _):
            for cc in range(D // L):
                sl2 = pl.ds(cc * L, L)
                plsc.addupdate(bufA[b].at[t, sl2], bufB[b][t, sl2])
            return ()
        lax.fori_loop(0, CH, add_body, (), unroll=2)

    def out_start(k, b):
        pltpu.async_copy(bufA[b], out_hbm.at[pl.ds(tok_base(k), CH), :],
                         semO[b])

    def out_wait(b):
        pltpu.make_async_copy(bufA[b], out_hbm.at[pl.ds(0, CH), :],
                              semO[b]).wait()

    fields_start(0, 0)

    @pl.loop(0, NCHUNK, step=2)
    def chunk_loop(c2):
        for b in (0, 1):
            k = c2 + b
            o = 1 - b
            fields_wait(b)
            idx_compute(b)

            @pl.when(k >= 2)
            def _():
                out_wait(b)

            gathers_start(b)

            @pl.when(k + 1 < NCHUNK)
            def _():
                fields_start(k + 1, o)

            @pl.when(k >= 1)
            def _():
                gathers_wait(o)
                accumulate(o)
                out_start(k - 1, o)

    gathers_wait(1)
    accumulate(1)
    out_start(NCHUNK - 1, 1)
    out_wait(0)
    out_wait(1)


def kernel(x, minute_W, hour_W, weekday_W, day_W, month_W, year_W):
    w1 = (minute_W[:, None, None, :] + weekday_W[None, :, None, :]
          + year_W[None, None, :, :]).reshape(V1, D)
    w2 = (hour_W[:, None, None, :] + day_W[None, :, None, :]
          + month_W[None, None, :, :]).reshape(V2, D)
    wc = jnp.concatenate([w1, w2], axis=0)
    xf = x.reshape(NTOK, NF).astype(jnp.int32).T
    out = _sc_embed(wc, xf)
    return out.reshape(B, S, D)
